# baseline (speedup 1.0000x reference)
import functools

import jax
import jax.numpy as jnp
from jax import lax
from jax.experimental import pallas as pl
from jax.experimental.pallas import tpu as pltpu
from jax.experimental.pallas import tpu_sc as plsc

PAD = 0
DIM = 32
L = 16
NC, NS = 2, 16
NW = NC * NS
ROWS, COLS = 16384, 50
XPW = ROWS // NW
NXR = 8
NCH = XPW // NXR
WINS = (0, 16, 34)


@functools.partial(
    pl.kernel,
    out_type=jax.ShapeDtypeStruct((ROWS, COLS, DIM), jnp.float32),
    mesh=plsc.VectorSubcoreMesh(core_axis_name="c", subcore_axis_name="s"),
    scratch_types=[
        pltpu.VMEM((XPW, COLS), jnp.int32),
        pltpu.VMEM((NXR, COLS, DIM), jnp.float32),
        pltpu.VMEM((NXR, COLS, DIM), jnp.float32),
        pltpu.SemaphoreType.DMA,
        pltpu.SemaphoreType.DMA,
        pltpu.SemaphoreType.DMA,
        pltpu.SemaphoreType.DMA,
    ],
    compiler_params=pltpu.CompilerParams(
        needs_layout_passes=False, use_tc_tiling_on_sc=False
    ),
)
def _emb(x_hbm, w_hbm, out_hbm, idx_v, rows0, rows1, sg0, sg1, so0, so1):
    wid = lax.axis_index("s") * NC + lax.axis_index("c")
    xrow0 = wid * XPW
    zv = jnp.zeros((L,), jnp.float32)
    iot = lax.iota(jnp.int32, L)

    pltpu.sync_copy(x_hbm.at[pl.ds(xrow0, XPW)], idx_v)

    def fire_gathers(g, rows, sem):
        for j in range(NXR):
            pltpu.async_copy(
                w_hbm.at[idx_v.at[g * NXR + j]], rows.at[j], sem
            )

    def drain_gathers(rows, sem):
        for j in range(NXR):
            pltpu.make_async_copy(
                w_hbm.at[pl.ds(0, COLS)], rows.at[j], sem
            ).wait()

    def fire_write(g, rows, sem):
        pltpu.async_copy(
            rows, out_hbm.at[pl.ds(xrow0 + g * NXR, NXR)], sem
        )

    def drain_write(rows, sem):
        pltpu.make_async_copy(
            rows, out_hbm.at[pl.ds(xrow0, NXR)], sem
        ).wait()

    def fixup(g, rows):
        def row_fix(r, _):
            irow = g * NXR + r
            vs = [idx_v[irow, pl.ds(w, L)] for w in WINS]
            mn = vs[0]
            for v in vs[1:]:
                mn = jnp.minimum(mn, v)
            for s in (8, 4, 2, 1):
                perm = (iot + s) % L
                rot = lax.gather(
                    mn,
                    perm[:, None],
                    lax.GatherDimensionNumbers(
                        offset_dims=(),
                        collapsed_slice_dims=(0,),
                        start_index_map=(0,),
                    ),
                    slice_sizes=(1,),
                    mode=lax.GatherScatterMode.PROMISE_IN_BOUNDS,
                )
                mn = jnp.minimum(mn, rot)

            @pl.when(mn[0] == PAD)
            def _():
                for q, w in enumerate(WINS):
                    zm = vs[q] == PAD
                    rv = jnp.full((L,), r, jnp.int32)
                    wp = w + iot
                    for col in range(DIM):
                        colv = jnp.full((L,), col, jnp.int32)
                        plsc.store_scatter(rows, [rv, wp, colv], zv, mask=zm)

            return 0

        lax.fori_loop(0, NXR, row_fix, 0)

    fire_gathers(0, rows0, sg0)

    def pair(i2, _):
        a = 2 * i2
        bq = a + 1
        c = a + 2

        @pl.when(i2 > 0)
        def _():
            drain_write(rows1, so1)

        fire_gathers(bq, rows1, sg1)
        drain_gathers(rows0, sg0)
        fixup(a, rows0)
        fire_write(a, rows0, so0)
        drain_gathers(rows1, sg1)
        fixup(bq, rows1)
        drain_write(rows0, so0)

        @pl.when(c < NCH)
        def _():
            fire_gathers(c, rows0, sg0)

        fire_write(bq, rows1, so1)
        return 0

    lax.fori_loop(0, NCH // 2, pair, 0)
    drain_write(rows1, so1)


def kernel(x, weight):
    return _emb(x.astype(jnp.int32), weight)


# speedup vs baseline: 1.0082x; 1.0082x over previous
import functools

import jax
import jax.numpy as jnp
from jax import lax
from jax.experimental import pallas as pl
from jax.experimental.pallas import tpu as pltpu
from jax.experimental.pallas import tpu_sc as plsc

PAD = 0
DIM = 32
L = 16
NC, NS = 2, 16
NW = NC * NS
ROWS, COLS = 16384, 50
XPW = ROWS // NW
NXR = 16
NCH = XPW // NXR
WINS = (0, 16, 34)


@functools.partial(
    pl.kernel,
    out_type=jax.ShapeDtypeStruct((ROWS, COLS, DIM), jnp.float32),
    mesh=plsc.VectorSubcoreMesh(core_axis_name="c", subcore_axis_name="s"),
    scratch_types=[
        pltpu.VMEM((XPW, COLS), jnp.int32),
        pltpu.VMEM((NXR, COLS, DIM), jnp.float32),
        pltpu.VMEM((NXR, COLS, DIM), jnp.float32),
        pltpu.SemaphoreType.DMA,
        pltpu.SemaphoreType.DMA,
        pltpu.SemaphoreType.DMA,
        pltpu.SemaphoreType.DMA,
    ],
    compiler_params=pltpu.CompilerParams(
        needs_layout_passes=False, use_tc_tiling_on_sc=False
    ),
)
def _emb(x_hbm, w_hbm, out_hbm, idx_v, rows0, rows1, sg0, sg1, so0, so1):
    wid = lax.axis_index("s") * NC + lax.axis_index("c")
    xrow0 = wid * XPW
    zv = jnp.zeros((L,), jnp.float32)
    iot = lax.iota(jnp.int32, L)

    pltpu.sync_copy(x_hbm.at[pl.ds(xrow0, XPW)], idx_v)

    def fire_gathers(g, rows, sem):
        for j in range(NXR):
            pltpu.async_copy(
                w_hbm.at[idx_v.at[g * NXR + j]], rows.at[j], sem
            )

    def drain_gathers(rows, sem):
        for j in range(NXR):
            pltpu.make_async_copy(
                w_hbm.at[pl.ds(0, COLS)], rows.at[j], sem
            ).wait()

    def fire_write(g, rows, sem):
        pltpu.async_copy(
            rows, out_hbm.at[pl.ds(xrow0 + g * NXR, NXR)], sem
        )

    def drain_write(rows, sem):
        pltpu.make_async_copy(
            rows, out_hbm.at[pl.ds(xrow0, NXR)], sem
        ).wait()

    def fixup(g, rows):
        def row_fix(r, _):
            irow = g * NXR + r
            vs = [idx_v[irow, pl.ds(w, L)] for w in WINS]
            mn = vs[0]
            for v in vs[1:]:
                mn = jnp.minimum(mn, v)
            for s in (8, 4, 2, 1):
                perm = (iot + s) % L
                rot = lax.gather(
                    mn,
                    perm[:, None],
                    lax.GatherDimensionNumbers(
                        offset_dims=(),
                        collapsed_slice_dims=(0,),
                        start_index_map=(0,),
                    ),
                    slice_sizes=(1,),
                    mode=lax.GatherScatterMode.PROMISE_IN_BOUNDS,
                )
                mn = jnp.minimum(mn, rot)

            @pl.when(mn[0] == PAD)
            def _():
                for q, w in enumerate(WINS):
                    zm = vs[q] == PAD
                    rv = jnp.full((L,), r, jnp.int32)
                    wp = w + iot
                    for col in range(DIM):
                        colv = jnp.full((L,), col, jnp.int32)
                        plsc.store_scatter(rows, [rv, wp, colv], zv, mask=zm)

            return 0

        lax.fori_loop(0, NXR, row_fix, 0)

    fire_gathers(0, rows0, sg0)

    def pair(i2, _):
        a = 2 * i2
        bq = a + 1
        c = a + 2

        @pl.when(i2 > 0)
        def _():
            drain_write(rows1, so1)

        fire_gathers(bq, rows1, sg1)
        drain_gathers(rows0, sg0)
        fixup(a, rows0)
        fire_write(a, rows0, so0)
        drain_gathers(rows1, sg1)
        fixup(bq, rows1)
        drain_write(rows0, so0)

        @pl.when(c < NCH)
        def _():
            fire_gathers(c, rows0, sg0)

        fire_write(bq, rows1, so1)
        return 0

    lax.fori_loop(0, NCH // 2, pair, 0)
    drain_write(rows1, so1)


def kernel(x, weight):
    return _emb(x.astype(jnp.int32), weight)
